# baseline (speedup 1.0000x reference)
import functools

import jax
import jax.numpy as jnp
from jax import lax
from jax.experimental import pallas as pl
from jax.experimental.pallas import tpu as pltpu
from jax.experimental.pallas import tpu_sc as plsc

_NC = 2
_NS = 16
_NW = _NC * _NS


def _tc_pack_wide(table):
    v, e = table.shape
    rows_in = 8000

    def body(t_ref, x_ref):
        x_ref[...] = jnp.pad(t_ref[...], ((0, 0), (0, e)))

    return pl.pallas_call(
        body,
        grid=(v // rows_in,),
        in_specs=[pl.BlockSpec((rows_in, e), lambda i: (i, 0))],
        out_specs=pl.BlockSpec((rows_in, 2 * e), lambda i: (i, 0)),
        out_shape=jax.ShapeDtypeStruct((v, 2 * e), jnp.float32),
    )(table)


def _sc_gather_wide(ids, x, chunk):
    n = ids.shape[0]
    d = x.shape[1]
    per_w = n // _NW
    n_chunks = per_w // chunk
    mesh = plsc.VectorSubcoreMesh(core_axis_name="c", subcore_axis_name="s")

    @functools.partial(
        pl.kernel,
        out_type=jax.ShapeDtypeStruct((n, d), jnp.float32),
        mesh=mesh,
        scratch_types=[
            pltpu.VMEM((chunk,), jnp.int32),
            pltpu.VMEM((chunk, d), jnp.float32),
            pltpu.SemaphoreType.DMA,
        ],
    )
    def k(ids_hbm, x_hbm, out_hbm, idx_v, rows_v, sem):
        wid = lax.axis_index("s") * _NC + lax.axis_index("c")
        base = wid * per_w

        def body(i, carry):
            off = base + i * chunk
            pltpu.sync_copy(ids_hbm.at[pl.ds(off, chunk)], idx_v)
            pltpu.async_copy(x_hbm.at[idx_v], rows_v, sem).wait()
            pltpu.sync_copy(rows_v, out_hbm.at[pl.ds(off, chunk)])
            return carry

        lax.fori_loop(0, n_chunks, body, 0)

    return k(ids, x)


def _tc_epilogue(rows, pos, gamma, beta, b, l, eps=1e-5):
    d2 = rows.shape[1]
    e = d2 // 2
    sb = 64

    def body(r_ref, pos_ref, g_ref, b_ref, o_ref):
        r3 = r_ref[...].reshape(sb, l, d2)
        x = r3[:, :, :e] + pos_ref[...]
        mean = jnp.mean(x, axis=-1, keepdims=True)
        xc = x - mean
        var = jnp.mean(xc * xc, axis=-1, keepdims=True)
        o_ref[...] = xc * (lax.rsqrt(var + eps) * g_ref[...]) + b_ref[...]

    return pl.pallas_call(
        body,
        grid=(b // sb,),
        in_specs=[
            pl.BlockSpec((sb * l, d2), lambda i: (i, 0)),
            pl.BlockSpec((1, l, e), lambda i: (0, 0, 0)),
            pl.BlockSpec((1, 1, e), lambda i: (0, 0, 0)),
            pl.BlockSpec((1, 1, e), lambda i: (0, 0, 0)),
        ],
        out_specs=pl.BlockSpec((sb, l, e), lambda i: (i, 0, 0)),
        out_shape=jax.ShapeDtypeStruct((b, l, e), jnp.float32),
    )(rows, pos.reshape(1, l, e), gamma.reshape(1, 1, e), beta.reshape(1, 1, e))


def kernel(input_ids, tok_table, pos_table, ln_gamma, ln_beta):
    b, l = input_ids.shape
    ids = input_ids.astype(jnp.int32).reshape(-1)
    x = _tc_pack_wide(tok_table)
    rows = _sc_gather_wide(ids, x, chunk=640)
    return _tc_epilogue(rows, pos_table[:l], ln_gamma, ln_beta, b, l)


# speedup vs baseline: 1.2005x; 1.2005x over previous
import functools

import jax
import jax.numpy as jnp
from jax import lax
from jax.experimental import pallas as pl
from jax.experimental.pallas import tpu as pltpu
from jax.experimental.pallas import tpu_sc as plsc

_NC = 2
_NS = 16
_NW = _NC * _NS


def _sc_gather(ids, table, chunk):
    n = ids.shape[0]
    d = table.shape[1]
    per_w = n // _NW
    n_chunks = per_w // chunk
    mesh = plsc.VectorSubcoreMesh(core_axis_name="c", subcore_axis_name="s")

    @functools.partial(
        pl.kernel,
        out_type=jax.ShapeDtypeStruct((n, d), jnp.float32),
        mesh=mesh,
        scratch_types=[
            pltpu.VMEM((chunk,), jnp.int32),
            pltpu.VMEM((chunk, d), jnp.float32),
            pltpu.SemaphoreType.DMA,
        ],
        compiler_params=pltpu.CompilerParams(use_tc_tiling_on_sc=False),
    )
    def k(ids_hbm, table_hbm, out_hbm, idx_v, rows_v, sem):
        wid = lax.axis_index("s") * _NC + lax.axis_index("c")
        base = wid * per_w

        def body(i, carry):
            off = base + i * chunk
            pltpu.sync_copy(ids_hbm.at[pl.ds(off, chunk)], idx_v)
            pltpu.async_copy(table_hbm.at[idx_v], rows_v, sem).wait()
            pltpu.sync_copy(rows_v, out_hbm.at[pl.ds(off, chunk)])
            return carry

        lax.fori_loop(0, n_chunks, body, 0)

    return k(ids, table)


def _tc_epilogue(rows2, pos_pair, gamma, beta, b, l, eps=1e-5):
    e = rows2.shape[1] // 2
    h = l // 2
    sb = 64
    rb = sb * h

    def body(r_ref, pos_ref, g_ref, b_ref, o_ref):
        lane = lax.broadcasted_iota(jnp.int32, (2 * e, 2), 0)
        col = lax.broadcasted_iota(jnp.int32, (2 * e, 2), 1)
        ones_lr = jnp.where((lane // e) == col, 1.0, 0.0)
        sel2 = ones_lr.T
        x = r_ref[...] + pos_ref[...]
        dn = (((1,), (0,)), ((), ()))
        s2 = lax.dot_general(x, ones_lr, dn)
        q2 = lax.dot_general(x * x, ones_lr, dn)
        mean2 = s2 * (1.0 / e)
        var2 = q2 * (1.0 / e) - mean2 * mean2
        rstd2 = lax.rsqrt(var2 + eps)
        mean = lax.dot_general(mean2, sel2, dn)
        rstd = lax.dot_general(rstd2, sel2, dn)
        y = (x - mean) * (rstd * g_ref[...]) + b_ref[...]
        o_ref[:, :h, :] = y[:, :e].reshape(sb, h, e)
        o_ref[:, h:, :] = y[:, e:].reshape(sb, h, e)

    return pl.pallas_call(
        body,
        grid=(b // sb,),
        in_specs=[
            pl.BlockSpec((rb, 2 * e), lambda i: (i, 0)),
            pl.BlockSpec((rb, 2 * e), lambda i: (0, 0)),
            pl.BlockSpec((1, 2 * e), lambda i: (0, 0)),
            pl.BlockSpec((1, 2 * e), lambda i: (0, 0)),
        ],
        out_specs=pl.BlockSpec((sb, l, e), lambda i: (i, 0, 0)),
        out_shape=jax.ShapeDtypeStruct((b, l, e), jnp.float32),
    )(rows2, pos_pair, jnp.concatenate([gamma, gamma]).reshape(1, 2 * e),
      jnp.concatenate([beta, beta]).reshape(1, 2 * e))


def kernel(input_ids, tok_table, pos_table, ln_gamma, ln_beta):
    b, l = input_ids.shape
    e = tok_table.shape[1]
    n = b * l
    h = l // 2
    ids2d = input_ids.astype(jnp.int32)
    ids_perm = ids2d.reshape(b, 2, h).transpose(0, 2, 1).reshape(-1)
    rows = _sc_gather(ids_perm, tok_table, chunk=800)
    rows2 = rows.reshape(n // 2, 2 * e)
    pos = pos_table[:l]
    pos_pair = jnp.tile(jnp.concatenate([pos[:h], pos[h:]], axis=1), (64, 1))
    return _tc_epilogue(rows2, pos_pair, ln_gamma, ln_beta, b, l)
